# Initial kernel scaffold; baseline (speedup 1.0000x reference)
import jax
import jax.numpy as jnp
from jax.experimental import pallas as pl


def kernel(hidden_states, gate_weight, gate_bias):
    raise NotImplementedError("write your pallas kernel here")


# speedup vs baseline: 1.9999x; 1.9999x over previous
import functools

import jax
import jax.numpy as jnp
from jax.experimental import pallas as pl
from jax.experimental.pallas import tpu as pltpu

D_MODEL = 768
N_EXPERTS = 64
BLOCK = 2048


def _router_body(x_ref, w_ref, b_ref, scores_ref, vals_ref, idx_ref):
    x = x_ref[...]
    w = w_ref[...]
    logits = jax.lax.dot_general(
        x, w, (((1,), (1,)), ((), ())), preferred_element_type=jnp.float32
    )
    logits = logits + b_ref[...]
    m = jnp.max(logits, axis=1, keepdims=True)
    e = jnp.exp(logits - m)
    s = jnp.sum(e, axis=1, keepdims=True)
    scores = e / s
    scores_ref[...] = scores

    iota = jax.lax.broadcasted_iota(jnp.int32, scores.shape, 1)
    m1 = jnp.max(scores, axis=1, keepdims=True)
    i1 = jnp.min(jnp.where(scores >= m1, iota, N_EXPERTS), axis=1, keepdims=True)
    masked = jnp.where(iota == i1, -jnp.inf, scores)
    m2 = jnp.max(masked, axis=1, keepdims=True)
    i2 = jnp.min(jnp.where(masked >= m2, iota, N_EXPERTS), axis=1, keepdims=True)
    vals_ref[...] = jnp.concatenate([m1, m2], axis=1)
    idx_ref[...] = jnp.concatenate([i1, i2], axis=1)


@jax.jit
def kernel(hidden_states, gate_weight, gate_bias):
    n_tokens = hidden_states.shape[0]
    grid = (n_tokens // BLOCK,)
    bias2d = gate_bias.reshape(1, N_EXPERTS)
    scores, vals, idx = pl.pallas_call(
        _router_body,
        grid=grid,
        in_specs=[
            pl.BlockSpec((BLOCK, D_MODEL), lambda i: (i, 0)),
            pl.BlockSpec((N_EXPERTS, D_MODEL), lambda i: (0, 0)),
            pl.BlockSpec((1, N_EXPERTS), lambda i: (0, 0)),
        ],
        out_specs=[
            pl.BlockSpec((BLOCK, N_EXPERTS), lambda i: (i, 0)),
            pl.BlockSpec((BLOCK, 2), lambda i: (i, 0)),
            pl.BlockSpec((BLOCK, 2), lambda i: (i, 0)),
        ],
        out_shape=[
            jax.ShapeDtypeStruct((n_tokens, N_EXPERTS), jnp.float32),
            jax.ShapeDtypeStruct((n_tokens, 2), jnp.float32),
            jax.ShapeDtypeStruct((n_tokens, 2), jnp.int32),
        ],
        compiler_params=pltpu.CompilerParams(
            dimension_semantics=("parallel",),
        ),
    )(hidden_states, gate_weight, bias2d)
    return ((idx, vals), scores)


# speedup vs baseline: 2.0464x; 1.0233x over previous
import functools

import jax
import jax.numpy as jnp
from jax.experimental import pallas as pl
from jax.experimental.pallas import tpu as pltpu

D_MODEL = 768
N_EXPERTS = 64
BLOCK = 2048


def _router_body(x_ref, w_ref, b_ref, scores_ref, vals_ref, idx_ref):
    x = x_ref[...]
    w = w_ref[...]
    logits = jax.lax.dot_general(
        x, w, (((1,), (1,)), ((), ())), preferred_element_type=jnp.float32
    )
    logits = logits + b_ref[...]
    m = jnp.max(logits, axis=1, keepdims=True)
    e = jnp.exp(logits - m)
    s = jnp.sum(e, axis=1, keepdims=True)
    scores = e / s
    scores_ref[...] = scores

    iota = jax.lax.broadcasted_iota(jnp.int32, scores.shape, 1)
    i1 = jnp.argmax(scores, axis=1)[:, None]
    m1 = jnp.max(scores, axis=1, keepdims=True)
    masked = jnp.where(iota == i1, -jnp.inf, scores)
    i2 = jnp.argmax(masked, axis=1)[:, None]
    m2 = jnp.max(masked, axis=1, keepdims=True)
    vals_ref[...] = jnp.concatenate([m1, m2], axis=1)
    idx_ref[...] = jnp.concatenate([i1, i2], axis=1)


@jax.jit
def kernel(hidden_states, gate_weight, gate_bias):
    n_tokens = hidden_states.shape[0]
    grid = (n_tokens // BLOCK,)
    bias2d = gate_bias.reshape(1, N_EXPERTS)
    scores, vals, idx = pl.pallas_call(
        _router_body,
        grid=grid,
        in_specs=[
            pl.BlockSpec((BLOCK, D_MODEL), lambda i: (i, 0)),
            pl.BlockSpec((N_EXPERTS, D_MODEL), lambda i: (0, 0)),
            pl.BlockSpec((1, N_EXPERTS), lambda i: (0, 0)),
        ],
        out_specs=[
            pl.BlockSpec((BLOCK, N_EXPERTS), lambda i: (i, 0)),
            pl.BlockSpec((BLOCK, 2), lambda i: (i, 0)),
            pl.BlockSpec((BLOCK, 2), lambda i: (i, 0)),
        ],
        out_shape=[
            jax.ShapeDtypeStruct((n_tokens, N_EXPERTS), jnp.float32),
            jax.ShapeDtypeStruct((n_tokens, 2), jnp.float32),
            jax.ShapeDtypeStruct((n_tokens, 2), jnp.int32),
        ],
        compiler_params=pltpu.CompilerParams(
            dimension_semantics=("parallel",),
        ),
    )(hidden_states, gate_weight, bias2d)
    return ((idx, vals), scores)


# speedup vs baseline: 2.1258x; 1.0388x over previous
import functools

import jax
import jax.numpy as jnp
from jax.experimental import pallas as pl
from jax.experimental.pallas import tpu as pltpu

D_MODEL = 768
N_EXPERTS = 64
BLOCK = 4096


def _router_body(x_ref, w_ref, b_ref, scores_ref, vals_ref, idx_ref):
    x = x_ref[...]
    w = w_ref[...]
    logits = jax.lax.dot_general(
        x, w, (((1,), (1,)), ((), ())), preferred_element_type=jnp.float32
    )
    logits = logits + b_ref[...]
    m = jnp.max(logits, axis=1, keepdims=True)
    e = jnp.exp(logits - m)
    s = jnp.sum(e, axis=1, keepdims=True)
    scores = e / s
    scores_ref[...] = scores

    iota = jax.lax.broadcasted_iota(jnp.int32, scores.shape, 1)
    i1 = jnp.argmax(scores, axis=1)[:, None]
    m1 = jnp.max(scores, axis=1, keepdims=True)
    masked = jnp.where(iota == i1, -jnp.inf, scores)
    i2 = jnp.argmax(masked, axis=1)[:, None]
    m2 = jnp.max(masked, axis=1, keepdims=True)
    vals_ref[...] = jnp.concatenate([m1, m2], axis=1)
    idx_ref[...] = jnp.concatenate([i1, i2], axis=1)


@jax.jit
def kernel(hidden_states, gate_weight, gate_bias):
    n_tokens = hidden_states.shape[0]
    grid = (n_tokens // BLOCK,)
    bias2d = gate_bias.reshape(1, N_EXPERTS)
    scores, vals, idx = pl.pallas_call(
        _router_body,
        grid=grid,
        in_specs=[
            pl.BlockSpec((BLOCK, D_MODEL), lambda i: (i, 0)),
            pl.BlockSpec((N_EXPERTS, D_MODEL), lambda i: (0, 0)),
            pl.BlockSpec((1, N_EXPERTS), lambda i: (0, 0)),
        ],
        out_specs=[
            pl.BlockSpec((BLOCK, N_EXPERTS), lambda i: (i, 0)),
            pl.BlockSpec((BLOCK, 2), lambda i: (i, 0)),
            pl.BlockSpec((BLOCK, 2), lambda i: (i, 0)),
        ],
        out_shape=[
            jax.ShapeDtypeStruct((n_tokens, N_EXPERTS), jnp.float32),
            jax.ShapeDtypeStruct((n_tokens, 2), jnp.float32),
            jax.ShapeDtypeStruct((n_tokens, 2), jnp.int32),
        ],
        compiler_params=pltpu.CompilerParams(
            dimension_semantics=("parallel",),
        ),
    )(hidden_states, gate_weight, bias2d)
    return ((idx, vals), scores)
